# baseline (speedup 1.0000x reference)
import functools

import jax
import jax.numpy as jnp
from jax import lax
from jax.experimental import pallas as pl
from jax.experimental.pallas import tpu as pltpu
from jax.experimental.pallas import tpu_sc as plsc

NC = 2
NS = 16
LANES = 16
NBUF = 4


def _comb_table(segment_embeddings, seq_len):
    n_seg, d = segment_embeddings.shape

    def body(seg_ref, out_ref):
        s_idx = lax.broadcasted_iota(jnp.int32, (seq_len, d), 0)
        d_idx = lax.broadcasted_iota(jnp.int32, (seq_len, d), 1)
        i2 = ((d_idx // 2) * 2).astype(jnp.float32)
        div = jnp.exp(-jnp.log(10000.0) * i2 / d)
        ang = s_idx.astype(jnp.float32) * div
        pe = jnp.where(d_idx % 2 == 0, jnp.sin(ang), jnp.cos(ang))
        for t in range(n_seg):
            out_ref[pl.ds(t * seq_len, seq_len), :] = pe + seg_ref[t:t + 1, :]

    return pl.pallas_call(
        body,
        out_shape=jax.ShapeDtypeStruct((n_seg * seq_len, d), jnp.float32),
    )(segment_embeddings)


def _sc_embed(tok_flat, seg_flat, word_embeddings, comb, seq_len):
    n = tok_flat.shape[0]
    d = word_embeddings.shape[1]
    n_seg = comb.shape[0] * 2 // (seq_len * d)
    nw = NC * NS
    rows_per_w = n // nw
    ch = 160
    n_chunks = rows_per_w // ch
    n_rounds, n_tail = divmod(n_chunks, NBUF)
    assert rows_per_w % ch == 0 and ch % 8 == 0 and n_chunks >= NBUF

    mesh = plsc.VectorSubcoreMesh(
        core_axis_name="c", subcore_axis_name="s",
        num_cores=NC, num_subcores=NS)

    @functools.partial(
        pl.kernel,
        out_type=jax.ShapeDtypeStruct((n, d), jnp.float32),
        mesh=mesh,
        compiler_params=pltpu.CompilerParams(needs_layout_passes=False),
        scratch_types=[
            pltpu.VMEM((rows_per_w,), jnp.int32),
            pltpu.VMEM((rows_per_w + LANES,), jnp.int32),
            pltpu.VMEM((n_seg * seq_len * d // 2,), jnp.int32),
            [pltpu.VMEM((ch, d), jnp.float32) for _ in range(NBUF)],
            [pltpu.SemaphoreType.DMA for _ in range(NBUF)],
            [pltpu.SemaphoreType.DMA for _ in range(NBUF)],
        ],
    )
    def k(tok_hbm, seg_hbm, wtab_hbm, comb_hbm, out_hbm,
          tokb, segb, combl, bufs, gsems, wsems):
        wid = lax.axis_index("s") * NC + lax.axis_index("c")
        base = wid * rows_per_w

        def start_gather(b, c):
            pltpu.async_copy(
                wtab_hbm.at[tokb.at[pl.ds(c * ch, ch)]], bufs[b], gsems[b])

        def wait_gather(b):
            pltpu.make_async_copy(
                wtab_hbm.at[pl.ds(0, ch)], bufs[b], gsems[b]).wait()

        def start_write(b, c):
            pltpu.async_copy(
                bufs[b], out_hbm.at[pl.ds(base + c * ch, ch)], wsems[b])

        def wait_write(b):
            pltpu.make_async_copy(
                bufs[b], out_hbm.at[pl.ds(base, ch)], wsems[b]).wait()

        pltpu.sync_copy(tok_hbm.at[pl.ds(base, rows_per_w)], tokb)
        for b in range(NBUF):
            start_gather(b, b)

        pltpu.sync_copy(seg_hbm.at[pl.ds(base, rows_per_w)],
                        segb.at[pl.ds(0, rows_per_w)])
        pltpu.sync_copy(comb_hbm, combl)

        def add_comb(b, c):
            buf = bufs[b]

            @plsc.parallel_loop(0, ch, unroll=8)
            def _(row):
                flat = c * ch + row
                seg_s = segb[pl.ds(flat, LANES)][0]
                coff = (seg_s * seq_len + lax.rem(flat, seq_len)) * (d // 2)
                for jj in range(d // (2 * LANES)):
                    w = combl[pl.ds(coff + jj * LANES, LANES)]
                    av = plsc.bitcast(w << 16, jnp.float32)
                    bv = plsc.bitcast(w & jnp.int32(-65536), jnp.float32)
                    plsc.addupdate(
                        buf.at[row, pl.ds(jj * 2 * LANES, LANES)], av)
                    plsc.addupdate(
                        buf.at[row, pl.ds(jj * 2 * LANES + LANES, LANES)], bv)

        def process(cc, b):
            prev = (b - 1) % NBUF

            def refill():
                wait_write(prev)
                start_gather(prev, cc + NBUF - 1)

            if isinstance(cc, int):
                if 1 <= cc and cc + NBUF - 1 < n_chunks:
                    refill()
            else:
                pl.when(jnp.logical_and(cc >= 1, cc + NBUF - 1 < n_chunks))(refill)

            wait_gather(b)
            add_comb(b, cc)
            start_write(b, cc)

        def round_body(rr, cc):
            for b in range(NBUF):
                process(rr * NBUF + b, b)
            return cc

        lax.fori_loop(0, n_rounds, round_body, 0)
        for t in range(n_tail):
            process(n_rounds * NBUF + t, t)

        for b in range(NBUF):
            wait_write(b)

    return k(tok_flat, seg_flat, word_embeddings, comb)


def kernel(input_tokens, input_seg, word_embeddings, segment_embeddings):
    b, s = input_tokens.shape
    d = word_embeddings.shape[1]
    comb = _comb_table(segment_embeddings, s)
    cb = comb.reshape(-1, d // 32, 2, 16).astype(jnp.bfloat16)
    cbb = jax.lax.bitcast_convert_type(cb, jnp.uint16).astype(jnp.uint32)
    words = cbb[:, :, 0, :] | (cbb[:, :, 1, :] << 16)
    comb = jax.lax.bitcast_convert_type(words, jnp.int32).reshape(-1)
    tok_flat = input_tokens.reshape(-1).astype(jnp.int32)
    seg_flat = input_seg.reshape(-1).astype(jnp.int32)
    out = _sc_embed(tok_flat, seg_flat, word_embeddings, comb, s)
    return out.reshape(b, s, d)


# speedup vs baseline: 1.0168x; 1.0168x over previous
import functools

import jax
import jax.numpy as jnp
from jax import lax
from jax.experimental import pallas as pl
from jax.experimental.pallas import tpu as pltpu
from jax.experimental.pallas import tpu_sc as plsc

NC = 2
NS = 16
LANES = 16
NBUF = 3


def _comb_table(segment_embeddings, seq_len):
    n_seg, d = segment_embeddings.shape

    def body(seg_ref, out_ref):
        s_idx = lax.broadcasted_iota(jnp.int32, (seq_len, d), 0)
        d_idx = lax.broadcasted_iota(jnp.int32, (seq_len, d), 1)
        i2 = ((d_idx // 2) * 2).astype(jnp.float32)
        div = jnp.exp(-jnp.log(10000.0) * i2 / d)
        ang = s_idx.astype(jnp.float32) * div
        pe = jnp.where(d_idx % 2 == 0, jnp.sin(ang), jnp.cos(ang))
        for t in range(n_seg):
            out_ref[pl.ds(t * seq_len, seq_len), :] = pe + seg_ref[t:t + 1, :]

    return pl.pallas_call(
        body,
        out_shape=jax.ShapeDtypeStruct((n_seg * seq_len, d), jnp.float32),
    )(segment_embeddings)


def _sc_embed(tok_flat, seg_flat, word_embeddings, comb, seq_len):
    n = tok_flat.shape[0]
    d = word_embeddings.shape[1]
    n_seg = comb.shape[0] * 2 // (seq_len * d)
    nw = NC * NS
    rows_per_w = n // nw
    ch = 200
    n_chunks = rows_per_w // ch
    n_rounds, n_tail = divmod(n_chunks, NBUF)
    assert rows_per_w % ch == 0 and ch % 8 == 0 and n_chunks >= NBUF

    mesh = plsc.VectorSubcoreMesh(
        core_axis_name="c", subcore_axis_name="s",
        num_cores=NC, num_subcores=NS)

    @functools.partial(
        pl.kernel,
        out_type=jax.ShapeDtypeStruct((n, d), jnp.float32),
        mesh=mesh,
        compiler_params=pltpu.CompilerParams(needs_layout_passes=False),
        scratch_types=[
            pltpu.VMEM((rows_per_w,), jnp.int32),
            pltpu.VMEM((rows_per_w + LANES,), jnp.int32),
            pltpu.VMEM((n_seg * seq_len * d // 2,), jnp.int32),
            [pltpu.VMEM((ch, d), jnp.float32) for _ in range(NBUF)],
            [pltpu.SemaphoreType.DMA for _ in range(NBUF)],
            [pltpu.SemaphoreType.DMA for _ in range(NBUF)],
        ],
    )
    def k(tok_hbm, seg_hbm, wtab_hbm, comb_hbm, out_hbm,
          tokb, segb, combl, bufs, gsems, wsems):
        wid = lax.axis_index("s") * NC + lax.axis_index("c")
        base = wid * rows_per_w

        def start_gather(b, c):
            pltpu.async_copy(
                wtab_hbm.at[tokb.at[pl.ds(c * ch, ch)]], bufs[b], gsems[b])

        def wait_gather(b):
            pltpu.make_async_copy(
                wtab_hbm.at[pl.ds(0, ch)], bufs[b], gsems[b]).wait()

        def start_write(b, c):
            pltpu.async_copy(
                bufs[b], out_hbm.at[pl.ds(base + c * ch, ch)], wsems[b])

        def wait_write(b):
            pltpu.make_async_copy(
                bufs[b], out_hbm.at[pl.ds(base, ch)], wsems[b]).wait()

        pltpu.sync_copy(tok_hbm.at[pl.ds(base, rows_per_w)], tokb)
        for b in range(NBUF):
            start_gather(b, b)

        pltpu.sync_copy(seg_hbm.at[pl.ds(base, rows_per_w)],
                        segb.at[pl.ds(0, rows_per_w)])
        pltpu.sync_copy(comb_hbm, combl)

        def add_comb(b, c):
            buf = bufs[b]

            @plsc.parallel_loop(0, ch, unroll=8)
            def _(row):
                flat = c * ch + row
                seg_s = segb[pl.ds(flat, LANES)][0]
                coff = (seg_s * seq_len + lax.rem(flat, seq_len)) * (d // 2)
                for jj in range(d // (2 * LANES)):
                    w = combl[pl.ds(coff + jj * LANES, LANES)]
                    av = plsc.bitcast(w << 16, jnp.float32)
                    bv = plsc.bitcast(w & jnp.int32(-65536), jnp.float32)
                    plsc.addupdate(
                        buf.at[row, pl.ds(jj * 2 * LANES, LANES)], av)
                    plsc.addupdate(
                        buf.at[row, pl.ds(jj * 2 * LANES + LANES, LANES)], bv)

        def process(cc, b):
            prev = (b - 1) % NBUF

            def refill():
                wait_write(prev)
                start_gather(prev, cc + NBUF - 1)

            if isinstance(cc, int):
                if 1 <= cc and cc + NBUF - 1 < n_chunks:
                    refill()
            else:
                pl.when(jnp.logical_and(cc >= 1, cc + NBUF - 1 < n_chunks))(refill)

            wait_gather(b)
            add_comb(b, cc)
            start_write(b, cc)

        def round_body(rr, cc):
            for b in range(NBUF):
                process(rr * NBUF + b, b)
            return cc

        lax.fori_loop(0, n_rounds, round_body, 0)
        for t in range(n_tail):
            process(n_rounds * NBUF + t, t)

        for b in range(NBUF):
            wait_write(b)

    return k(tok_flat, seg_flat, word_embeddings, comb)


def kernel(input_tokens, input_seg, word_embeddings, segment_embeddings):
    b, s = input_tokens.shape
    d = word_embeddings.shape[1]
    comb = _comb_table(segment_embeddings, s)
    cb = comb.reshape(-1, d // 32, 2, 16).astype(jnp.bfloat16)
    cbb = jax.lax.bitcast_convert_type(cb, jnp.uint16).astype(jnp.uint32)
    words = cbb[:, :, 0, :] | (cbb[:, :, 1, :] << 16)
    comb = jax.lax.bitcast_convert_type(words, jnp.int32).reshape(-1)
    tok_flat = input_tokens.reshape(-1).astype(jnp.int32)
    seg_flat = input_seg.reshape(-1).astype(jnp.int32)
    out = _sc_embed(tok_flat, seg_flat, word_embeddings, comb, s)
    return out.reshape(b, s, d)


# speedup vs baseline: 1.0185x; 1.0016x over previous
import functools

import jax
import jax.numpy as jnp
import ml_dtypes
import numpy as np
from jax import lax
from jax.experimental import pallas as pl
from jax.experimental.pallas import tpu as pltpu
from jax.experimental.pallas import tpu_sc as plsc

NC = 2
NS = 16
LANES = 16
NBUF = 3


def _pe_words(seq_len, d):
    pos = np.arange(seq_len, dtype=np.float32)[:, None]
    i = np.arange(0, d, 2, dtype=np.float32)
    div_term = np.exp(-np.log(10000.0) * i / d)
    angles = pos * div_term[None, :]
    pe = np.zeros((seq_len, d), dtype=np.float32)
    pe[:, 0::2] = np.sin(angles)
    pe[:, 1::2] = np.cos(angles)
    bits = pe.astype(ml_dtypes.bfloat16).view(np.uint16)
    bits = bits.astype(np.uint32).reshape(seq_len, d // 32, 2, 16)
    words = bits[:, :, 0, :] | (bits[:, :, 1, :] << 16)
    return words.astype(np.int32).reshape(-1)


def _sc_embed(tok_flat, seg_flat, word_embeddings, pe_words, seg_emb, seq_len):
    n = tok_flat.shape[0]
    d = word_embeddings.shape[1]
    nw = NC * NS
    rows_per_w = n // nw
    ch = 200
    n_chunks = rows_per_w // ch
    n_rounds, n_tail = divmod(n_chunks, NBUF)
    assert rows_per_w % ch == 0 and ch % 8 == 0 and n_chunks >= NBUF

    mesh = plsc.VectorSubcoreMesh(
        core_axis_name="c", subcore_axis_name="s",
        num_cores=NC, num_subcores=NS)

    @functools.partial(
        pl.kernel,
        out_type=jax.ShapeDtypeStruct((n, d), jnp.float32),
        mesh=mesh,
        compiler_params=pltpu.CompilerParams(needs_layout_passes=False),
        scratch_types=[
            pltpu.VMEM((rows_per_w,), jnp.int32),
            pltpu.VMEM((rows_per_w + LANES,), jnp.int32),
            pltpu.VMEM((seq_len * d // 2,), jnp.int32),
            pltpu.VMEM(seg_emb.shape, jnp.float32),
            [pltpu.VMEM((ch, d), jnp.float32) for _ in range(NBUF)],
            [pltpu.SemaphoreType.DMA for _ in range(NBUF)],
            [pltpu.SemaphoreType.DMA for _ in range(NBUF)],
        ],
    )
    def k(tok_hbm, seg_hbm, wtab_hbm, pe_hbm, segemb_hbm, out_hbm,
          tokb, segb, pel, segl, bufs, gsems, wsems):
        wid = lax.axis_index("s") * NC + lax.axis_index("c")
        base = wid * rows_per_w

        def start_gather(b, c):
            pltpu.async_copy(
                wtab_hbm.at[tokb.at[pl.ds(c * ch, ch)]], bufs[b], gsems[b])

        def wait_gather(b):
            pltpu.make_async_copy(
                wtab_hbm.at[pl.ds(0, ch)], bufs[b], gsems[b]).wait()

        def start_write(b, c):
            pltpu.async_copy(
                bufs[b], out_hbm.at[pl.ds(base + c * ch, ch)], wsems[b])

        def wait_write(b):
            pltpu.make_async_copy(
                bufs[b], out_hbm.at[pl.ds(base, ch)], wsems[b]).wait()

        pltpu.sync_copy(tok_hbm.at[pl.ds(base, rows_per_w)], tokb)
        for b in range(NBUF):
            start_gather(b, b)

        pltpu.sync_copy(seg_hbm.at[pl.ds(base, rows_per_w)],
                        segb.at[pl.ds(0, rows_per_w)])
        pltpu.sync_copy(pe_hbm, pel)
        pltpu.sync_copy(segemb_hbm, segl)
        ev = [[segl[t, pl.ds(j * LANES, LANES)] for j in range(d // LANES)]
              for t in range(seg_emb.shape[0])]

        def add_pe_seg(b, c):
            buf = bufs[b]

            @plsc.parallel_loop(0, ch, unroll=8)
            def _(row):
                flat = c * ch + row
                seg_s = segb[pl.ds(flat, LANES)][0]
                poff = lax.rem(flat, seq_len) * (d // 2)
                is1 = seg_s == 1
                for jj in range(d // (2 * LANES)):
                    w = pel[pl.ds(poff + jj * LANES, LANES)]
                    av = plsc.bitcast(w << 16, jnp.float32)
                    bv = plsc.bitcast(w & jnp.int32(-65536), jnp.float32)
                    sa = jnp.where(is1, ev[1][2 * jj], ev[0][2 * jj])
                    sb = jnp.where(is1, ev[1][2 * jj + 1], ev[0][2 * jj + 1])
                    plsc.addupdate(
                        buf.at[row, pl.ds(jj * 2 * LANES, LANES)], av + sa)
                    plsc.addupdate(
                        buf.at[row, pl.ds(jj * 2 * LANES + LANES, LANES)],
                        bv + sb)

        def process(cc, b):
            prev = (b - 1) % NBUF

            def refill():
                wait_write(prev)
                start_gather(prev, cc + NBUF - 1)

            if isinstance(cc, int):
                if 1 <= cc and cc + NBUF - 1 < n_chunks:
                    refill()
            else:
                pl.when(jnp.logical_and(cc >= 1, cc + NBUF - 1 < n_chunks))(refill)

            wait_gather(b)
            add_pe_seg(b, cc)
            start_write(b, cc)

        def round_body(rr, cc):
            for b in range(NBUF):
                process(rr * NBUF + b, b)
            return cc

        lax.fori_loop(0, n_rounds, round_body, 0)
        for t in range(n_tail):
            process(n_rounds * NBUF + t, t)

        for b in range(NBUF):
            wait_write(b)

    return k(tok_flat, seg_flat, word_embeddings, pe_words, seg_emb)


def kernel(input_tokens, input_seg, word_embeddings, segment_embeddings):
    b, s = input_tokens.shape
    pe_words = jnp.asarray(_pe_words(s, word_embeddings.shape[1]))
    tok_flat = input_tokens.reshape(-1).astype(jnp.int32)
    seg_flat = input_seg.reshape(-1).astype(jnp.int32)
    out = _sc_embed(tok_flat, seg_flat, word_embeddings, pe_words,
                    segment_embeddings, s)
    return out.reshape(b, s, word_embeddings.shape[1])
